# baseline (speedup 1.0000x reference)
import functools

import jax
import jax.numpy as jnp
from jax.experimental import pallas as pl
from jax.experimental.pallas import tpu as pltpu


LANE = 128
TM = 256
SRC_BITS = 13
SENT = 1 << 22


def _round_up(x, m):
    return (x + m - 1) // m * m


def _pad2(a, rows, cols):
    pr, pc = rows - a.shape[0], cols - a.shape[1]
    if pr == 0 and pc == 0:
        return a
    return jnp.pad(a, ((0, pr), (0, pc)))


def _pos_kernel(key_ref, triu_ref, pos_ref, co_ref, carry_ref, poffs_ref,
                *, eb, nblk):
    ph = pl.program_id(0)
    j = pl.program_id(1)

    @pl.when((ph == 0) & (j == 0))
    def _():
        carry_ref[...] = jnp.zeros_like(carry_ref)

    key_row = key_ref[0]
    iotc = jax.lax.broadcasted_iota(jnp.int32, (nblk, 1), 0)
    mj = jnp.where(iotc == key_row, 1.0, 0.0)
    p1 = jnp.dot(mj, triu_ref[...], preferred_element_type=jnp.float32)

    @pl.when(ph == 0)
    def _():
        carry_ref[...] += p1[:, LANE - 1:LANE]

        @pl.when(j == eb - 1)
        def _():
            sizes = carry_ref[...]
            nch = jnp.maximum(jnp.floor((sizes + (TM - 1)) / TM), 1.0)
            i0 = jax.lax.broadcasted_iota(jnp.int32, (nblk, nblk), 0)
            i1 = jax.lax.broadcasted_iota(jnp.int32, (nblk, nblk), 1)
            tril_s = jnp.where(i0 > i1, 1.0, 0.0)
            tril_i = jnp.where(i0 >= i1, 1.0, 0.0)
            poffs_ref[...] = jnp.dot(tril_s, nch,
                                     preferred_element_type=jnp.float32) * TM
            cum = jnp.dot(tril_i, nch, preferred_element_type=jnp.float32)
            co_ref[...] = jnp.concatenate(
                [jnp.zeros((1, 1), jnp.float32), cum], axis=0).astype(
                    jnp.int32)
            carry_ref[...] = jnp.zeros_like(carry_ref)

    @pl.when(ph == 1)
    def _():
        val = p1 + carry_ref[...] + poffs_ref[...] - 1.0
        pos_row = jnp.sum(mj * val, axis=0, keepdims=True)
        pos_ref[...] = pos_row.astype(jnp.int32).reshape(1, 1, LANE)
        carry_ref[...] += p1[:, LANE - 1:LANE]


def _deg_kernel(co_ref, pad_ref, o_ref, *, n):
    i = pl.program_id(0)
    o_ref[...] = jnp.zeros_like(o_ref)
    c0 = co_ref[i]
    c1 = co_ref[i + 1]

    def chunk(c, _):
        dstl = pad_ref[pl.ds(c, 1), 0, :] >> SRC_BITS
        iot = jax.lax.broadcasted_iota(jnp.int32, (TM, TM), 0)
        dt = jnp.where(iot == dstl, 1.0, 0.0)
        o_ref[...] += jnp.sum(dt, axis=1, keepdims=True)
        return 0

    jax.lax.fori_loop(c0, c1, chunk, 0)

    row = i * TM + jax.lax.broadcasted_iota(jnp.int32, (TM, 1), 0)
    deg = o_ref[...] + 1.0
    o_ref[...] = jnp.where(row < n, 1.0 / jnp.sqrt(deg), 0.0)


def _proj_kernel(x_ref, w_ref, d_ref, o_ref):
    xb = x_ref[...].astype(jnp.bfloat16)
    acc = jnp.dot(xb, w_ref[...], preferred_element_type=jnp.float32)
    o_ref[...] = acc * d_ref[...]


def _agg_kernel(co_ref, pad_sm_ref, pad_vm_ref, s_ref, d_ref, b_ref, w2_ref,
                o_ref, acc_ref, g_ref, g2_ref, g3_ref, g4_ref, *, last):
    i = pl.program_id(0)
    acc_ref[...] = jnp.zeros_like(acc_ref)
    c0 = co_ref[i]
    c1 = co_ref[i + 1]

    iot = jax.lax.broadcasted_iota(jnp.int32, (TM, TM), 0)
    msk = (1 << SRC_BITS) - 1

    def onehot(c):
        dstl = pad_vm_ref[pl.ds(c, 1), 0, :] >> SRC_BITS
        return jnp.where(iot == dstl, 1.0, 0.0).astype(jnp.float32)

    grefs = (g_ref, g2_ref, g3_ref, g4_ref)

    def quad(j, _):
        cbs = [(c0 + 4 * j + q) * TM for q in range(4)]
        for e in range(TM):
            for q in range(4):
                s0 = pad_sm_ref[cbs[q] + e] & msk
                grefs[q][pl.ds(e, 1), :] = s_ref[pl.ds(s0, 1), :]
        for q in range(4):
            acc_ref[...] += jnp.dot(onehot(c0 + 4 * j + q), grefs[q][...],
                                    preferred_element_type=jnp.float32)
        return 0

    jax.lax.fori_loop(0, (c1 - c0) // 4, quad, 0)

    def tail(c, _):
        for e in range(TM):
            srcv = pad_sm_ref[c * TM + e] & msk
            g_ref[pl.ds(e, 1), :] = s_ref[pl.ds(srcv, 1), :]
        acc_ref[...] += jnp.dot(onehot(c), g_ref[...],
                                preferred_element_type=jnp.float32)
        return 0

    jax.lax.fori_loop(c0 + ((c1 - c0) // 4) * 4, c1, tail, 0)

    acc = acc_ref[...] + s_ref[pl.ds(i * TM, TM), :]
    if last:
        o_ref[...] = acc * d_ref[...] + b_ref[...]
    else:
        h = jnp.maximum(acc * d_ref[...] + b_ref[...], 0.0)
        m2 = jnp.dot(h.astype(jnp.bfloat16), w2_ref[...],
                     preferred_element_type=jnp.float32)
        o_ref[...] = m2 * d_ref[...]


def _pos(key3, nblk):
    eb = key3.shape[0]
    return pl.pallas_call(
        functools.partial(_pos_kernel, eb=eb, nblk=nblk),
        out_shape=(
            jax.ShapeDtypeStruct((eb, 1, LANE), jnp.int32),
            jax.ShapeDtypeStruct((nblk + 1, 1), jnp.int32),
        ),
        grid=(2, eb),
        in_specs=[
            pl.BlockSpec((1, 1, LANE), lambda p, j: (j, 0, 0)),
            pl.BlockSpec((LANE, LANE), lambda p, j: (0, 0)),
        ],
        out_specs=(
            pl.BlockSpec((1, 1, LANE), lambda p, j: (p * j, 0, 0)),
            pl.BlockSpec((nblk + 1, 1), lambda p, j: (0, 0)),
        ),
        scratch_shapes=[
            pltpu.VMEM((nblk, 1), jnp.float32),
            pltpu.VMEM((nblk, 1), jnp.float32),
        ],
        compiler_params=pltpu.CompilerParams(
            dimension_semantics=("arbitrary", "arbitrary")),
    )(key3, jnp.triu(jnp.ones((LANE, LANE), jnp.float32)))


def _deg(co33, padded3, n_pad, n):
    nc = padded3.shape[0]
    return pl.pallas_call(
        functools.partial(_deg_kernel, n=n),
        out_shape=jax.ShapeDtypeStruct((n_pad, 1), jnp.float32),
        grid=(n_pad // TM,),
        in_specs=[
            pl.BlockSpec(memory_space=pltpu.SMEM),
            pl.BlockSpec((nc, 1, TM), lambda i: (0, 0, 0)),
        ],
        out_specs=pl.BlockSpec((TM, 1), lambda i: (i, 0)),
        compiler_params=pltpu.CompilerParams(
            dimension_semantics=("arbitrary",)),
    )(co33, padded3)


def _proj(x_p, w1b, dinv):
    n_pad, f_in_pad = x_p.shape
    hid_pad = w1b.shape[1]
    return pl.pallas_call(
        _proj_kernel,
        out_shape=jax.ShapeDtypeStruct((n_pad, hid_pad), jnp.float32),
        grid=(n_pad // TM,),
        in_specs=[
            pl.BlockSpec((TM, f_in_pad), lambda i: (i, 0)),
            pl.BlockSpec((f_in_pad, hid_pad), lambda i: (0, 0)),
            pl.BlockSpec((TM, 1), lambda i: (i, 0)),
        ],
        out_specs=pl.BlockSpec((TM, hid_pad), lambda i: (i, 0)),
        compiler_params=pltpu.CompilerParams(
            dimension_semantics=("parallel",)),
    )(x_p, w1b, dinv)


def _agg(co33, padded_flat, padded3, s_full, dinv, bias, w2b, *, last,
         out_cols):
    n_pad = s_full.shape[0]
    cols = s_full.shape[1]
    nc = padded3.shape[0]
    body = functools.partial(_agg_kernel, last=last)
    return pl.pallas_call(
        body,
        out_shape=jax.ShapeDtypeStruct((n_pad, out_cols), jnp.float32),
        grid=(n_pad // TM,),
        in_specs=[
            pl.BlockSpec(memory_space=pltpu.SMEM),
            pl.BlockSpec(memory_space=pltpu.SMEM),
            pl.BlockSpec((nc, 1, TM), lambda i: (0, 0, 0)),
            pl.BlockSpec((n_pad, cols), lambda i: (0, 0)),
            pl.BlockSpec((TM, 1), lambda i: (i, 0)),
            pl.BlockSpec((1, bias.shape[1]), lambda i: (0, 0)),
            pl.BlockSpec((w2b.shape[0], w2b.shape[1]), lambda i: (0, 0)),
        ],
        out_specs=pl.BlockSpec((TM, out_cols), lambda i: (i, 0)),
        scratch_shapes=[
            pltpu.VMEM((TM, cols), jnp.float32),
            pltpu.VMEM((TM, cols), jnp.float32),
            pltpu.VMEM((TM, cols), jnp.float32),
            pltpu.VMEM((TM, cols), jnp.float32),
            pltpu.VMEM((TM, cols), jnp.float32),
        ],
        compiler_params=pltpu.CompilerParams(
            dimension_semantics=("arbitrary",)),
    )(co33, padded_flat, padded3, s_full, dinv, bias, w2b)


def kernel(x, edge_index, w1, b1, w2, b2):
    n, f_in = x.shape
    hid = w1.shape[1]
    f_out = w2.shape[1]

    n_pad = _round_up(n, TM)
    f_in_pad = _round_up(f_in, LANE)
    hid_pad = _round_up(hid, LANE)
    f_out_pad = _round_up(f_out, LANE)
    nblk = n_pad // TM

    src = edge_index[0].astype(jnp.int32)
    dst = edge_index[1].astype(jnp.int32)
    ne = src.shape[0]
    er = _round_up(ne, LANE)
    eb = er // LANE
    p_max = _round_up(ne, TM) + nblk * TM
    nc_max = p_max // TM

    key = jnp.pad(dst // TM, (0, er - ne), constant_values=-1)
    pos3, co2 = _pos(key.reshape(eb, 1, LANE), nblk)
    co = co2.reshape(-1)

    pos = pos3.reshape(-1)
    valid = jnp.arange(er, dtype=jnp.int32) < ne
    pos = jnp.where(valid, pos, p_max)

    dstl = dst % TM
    packed = jnp.pad(src, (0, er - ne)) | (jnp.pad(dstl, (0, er - ne))
                                           << SRC_BITS)
    padded = jnp.full((p_max,), SENT, jnp.int32).at[pos].set(packed)
    padded3 = padded.reshape(nc_max, 1, TM)

    x_p = _pad2(x, n_pad, f_in_pad)
    w1b = _pad2(w1, f_in_pad, hid_pad).astype(jnp.bfloat16)
    w2b = _pad2(w2, hid_pad, f_out_pad).astype(jnp.bfloat16)
    b1_p = _pad2(b1.reshape(1, -1), 1, hid_pad)
    b2_p = _pad2(b2.reshape(1, -1), 1, f_out_pad)

    dinv = _deg(co, padded3, n_pad, n)
    s1 = _proj(x_p, w1b, dinv)
    m2 = _agg(co, padded, padded3, s1, dinv, b1_p, w2b, last=False,
              out_cols=f_out_pad)
    out_p = _agg(co, padded, padded3, m2, dinv, b2_p, w2b, last=True,
                 out_cols=f_out_pad)

    return out_p[:n, :f_out]


# speedup vs baseline: 1.4056x; 1.4056x over previous
import functools

import jax
import jax.numpy as jnp
from jax.experimental import pallas as pl
from jax.experimental.pallas import tpu as pltpu


LANE = 128
TM = 256
SRC_BITS = 13
SENT = 1 << 22


def _round_up(x, m):
    return (x + m - 1) // m * m


def _pad2(a, rows, cols):
    pr, pc = rows - a.shape[0], cols - a.shape[1]
    if pr == 0 and pc == 0:
        return a
    return jnp.pad(a, ((0, pr), (0, pc)))


def _pos_kernel(key_ref, triu_ref, pos_ref, co_ref, carry_ref, poffs_ref,
                *, nsteps, nblk, rows):
    ph = pl.program_id(0)
    j = pl.program_id(1)

    @pl.when((ph == 0) & (j == 0))
    def _():
        carry_ref[...] = jnp.zeros_like(carry_ref)

    iotc = jax.lax.broadcasted_iota(jnp.int32, (nblk, 1), 0)
    for r in range(rows):
        key_row = key_ref[r]
        mj = jnp.where(iotc == key_row, 1.0, 0.0)
        p1 = jnp.dot(mj, triu_ref[...], preferred_element_type=jnp.float32)

        @pl.when(ph == 1)
        def _():
            val = p1 + carry_ref[...] + poffs_ref[...] - 1.0
            pos_row = jnp.sum(mj * val, axis=0, keepdims=True)
            pos_ref[r] = pos_row.astype(jnp.int32)

        carry_ref[...] += p1[:, LANE - 1:LANE]

    @pl.when((ph == 0) & (j == nsteps - 1))
    def _():
        sizes = carry_ref[...]
        nch = jnp.maximum(jnp.floor((sizes + (TM - 1)) / TM), 1.0)
        i0 = jax.lax.broadcasted_iota(jnp.int32, (nblk, nblk), 0)
        i1 = jax.lax.broadcasted_iota(jnp.int32, (nblk, nblk), 1)
        tril_s = jnp.where(i0 > i1, 1.0, 0.0)
        tril_i = jnp.where(i0 >= i1, 1.0, 0.0)
        poffs_ref[...] = jnp.dot(tril_s, nch,
                                 preferred_element_type=jnp.float32) * TM
        cum = jnp.dot(tril_i, nch, preferred_element_type=jnp.float32)
        co_ref[...] = jnp.concatenate(
            [jnp.zeros((1, 1), jnp.float32), cum], axis=0).astype(jnp.int32)
        carry_ref[...] = jnp.zeros_like(carry_ref)


def _deg_kernel(co_ref, pad_ref, o_ref, *, n):
    i = pl.program_id(0)
    o_ref[...] = jnp.zeros_like(o_ref)
    c0 = co_ref[i]
    c1 = co_ref[i + 1]

    def chunk(c, _):
        dstl = pad_ref[pl.ds(c, 1), 0, :] >> SRC_BITS
        iot = jax.lax.broadcasted_iota(jnp.int32, (TM, TM), 0)
        dt = jnp.where(iot == dstl, 1.0, 0.0)
        o_ref[...] += jnp.sum(dt, axis=1, keepdims=True)
        return 0

    jax.lax.fori_loop(c0, c1, chunk, 0)

    row = i * TM + jax.lax.broadcasted_iota(jnp.int32, (TM, 1), 0)
    deg = o_ref[...] + 1.0
    o_ref[...] = jnp.where(row < n, 1.0 / jnp.sqrt(deg), 0.0)


def _proj_kernel(x_ref, w_ref, d_ref, o_ref):
    xb = x_ref[...].astype(jnp.bfloat16)
    acc = jnp.dot(xb, w_ref[...], preferred_element_type=jnp.float32)
    o_ref[...] = acc * d_ref[...]


def _agg_kernel(co_ref, pad_sm_ref, pad_vm_ref, s_ref, d_ref, b_ref, w2_ref,
                o_ref, acc_ref, g_ref, g2_ref, g3_ref, g4_ref, *, last):
    i = pl.program_id(0)
    acc_ref[...] = jnp.zeros_like(acc_ref)
    c0 = co_ref[i]
    c1 = co_ref[i + 1]

    iot = jax.lax.broadcasted_iota(jnp.int32, (TM, TM), 0)
    msk = (1 << SRC_BITS) - 1

    def onehot(c):
        dstl = pad_vm_ref[pl.ds(c, 1), 0, :] >> SRC_BITS
        return jnp.where(iot == dstl, 1.0, 0.0).astype(jnp.float32)

    grefs = (g_ref, g2_ref, g3_ref, g4_ref)

    def quad(j, _):
        cbs = [(c0 + 4 * j + q) * TM for q in range(4)]
        for e in range(TM):
            for q in range(4):
                s0 = pad_sm_ref[cbs[q] + e] & msk
                grefs[q][pl.ds(e, 1), :] = s_ref[pl.ds(s0, 1), :]
        for q in range(4):
            acc_ref[...] += jnp.dot(onehot(c0 + 4 * j + q), grefs[q][...],
                                    preferred_element_type=jnp.float32)
        return 0

    jax.lax.fori_loop(0, (c1 - c0) // 4, quad, 0)

    def tail(c, _):
        for e in range(TM):
            srcv = pad_sm_ref[c * TM + e] & msk
            g_ref[pl.ds(e, 1), :] = s_ref[pl.ds(srcv, 1), :]
        acc_ref[...] += jnp.dot(onehot(c), g_ref[...],
                                preferred_element_type=jnp.float32)
        return 0

    jax.lax.fori_loop(c0 + ((c1 - c0) // 4) * 4, c1, tail, 0)

    acc = acc_ref[...] + s_ref[pl.ds(i * TM, TM), :]
    if last:
        o_ref[...] = acc * d_ref[...] + b_ref[...]
    else:
        h = jnp.maximum(acc * d_ref[...] + b_ref[...], 0.0)
        m2 = jnp.dot(h.astype(jnp.bfloat16), w2_ref[...],
                     preferred_element_type=jnp.float32)
        o_ref[...] = m2 * d_ref[...]


def _pos(key3, nblk, rows=40):
    eb = key3.shape[0]
    while eb % rows != 0:
        rows -= 1
    nsteps = eb // rows
    return pl.pallas_call(
        functools.partial(_pos_kernel, nsteps=nsteps, nblk=nblk, rows=rows),
        out_shape=(
            jax.ShapeDtypeStruct((eb, 1, LANE), jnp.int32),
            jax.ShapeDtypeStruct((nblk + 1, 1), jnp.int32),
        ),
        grid=(2, nsteps),
        in_specs=[
            pl.BlockSpec((rows, 1, LANE), lambda p, j: (j, 0, 0)),
            pl.BlockSpec((LANE, LANE), lambda p, j: (0, 0)),
        ],
        out_specs=(
            pl.BlockSpec((rows, 1, LANE), lambda p, j: (p * j, 0, 0)),
            pl.BlockSpec((nblk + 1, 1), lambda p, j: (0, 0)),
        ),
        scratch_shapes=[
            pltpu.VMEM((nblk, 1), jnp.float32),
            pltpu.VMEM((nblk, 1), jnp.float32),
        ],
        compiler_params=pltpu.CompilerParams(
            dimension_semantics=("arbitrary", "arbitrary")),
    )(key3, jnp.triu(jnp.ones((LANE, LANE), jnp.float32)))


def _deg(co33, padded3, n_pad, n):
    nc = padded3.shape[0]
    return pl.pallas_call(
        functools.partial(_deg_kernel, n=n),
        out_shape=jax.ShapeDtypeStruct((n_pad, 1), jnp.float32),
        grid=(n_pad // TM,),
        in_specs=[
            pl.BlockSpec(memory_space=pltpu.SMEM),
            pl.BlockSpec((nc, 1, TM), lambda i: (0, 0, 0)),
        ],
        out_specs=pl.BlockSpec((TM, 1), lambda i: (i, 0)),
        compiler_params=pltpu.CompilerParams(
            dimension_semantics=("arbitrary",)),
    )(co33, padded3)


def _proj(x_p, w1b, dinv):
    n_pad, f_in_pad = x_p.shape
    hid_pad = w1b.shape[1]
    return pl.pallas_call(
        _proj_kernel,
        out_shape=jax.ShapeDtypeStruct((n_pad, hid_pad), jnp.float32),
        grid=(n_pad // TM,),
        in_specs=[
            pl.BlockSpec((TM, f_in_pad), lambda i: (i, 0)),
            pl.BlockSpec((f_in_pad, hid_pad), lambda i: (0, 0)),
            pl.BlockSpec((TM, 1), lambda i: (i, 0)),
        ],
        out_specs=pl.BlockSpec((TM, hid_pad), lambda i: (i, 0)),
        compiler_params=pltpu.CompilerParams(
            dimension_semantics=("parallel",)),
    )(x_p, w1b, dinv)


def _agg(co33, padded_flat, padded3, s_full, dinv, bias, w2b, *, last,
         out_cols):
    n_pad = s_full.shape[0]
    cols = s_full.shape[1]
    nc = padded3.shape[0]
    body = functools.partial(_agg_kernel, last=last)
    return pl.pallas_call(
        body,
        out_shape=jax.ShapeDtypeStruct((n_pad, out_cols), jnp.float32),
        grid=(n_pad // TM,),
        in_specs=[
            pl.BlockSpec(memory_space=pltpu.SMEM),
            pl.BlockSpec(memory_space=pltpu.SMEM),
            pl.BlockSpec((nc, 1, TM), lambda i: (0, 0, 0)),
            pl.BlockSpec((n_pad, cols), lambda i: (0, 0)),
            pl.BlockSpec((TM, 1), lambda i: (i, 0)),
            pl.BlockSpec((1, bias.shape[1]), lambda i: (0, 0)),
            pl.BlockSpec((w2b.shape[0], w2b.shape[1]), lambda i: (0, 0)),
        ],
        out_specs=pl.BlockSpec((TM, out_cols), lambda i: (i, 0)),
        scratch_shapes=[
            pltpu.VMEM((TM, cols), jnp.float32),
            pltpu.VMEM((TM, cols), jnp.float32),
            pltpu.VMEM((TM, cols), jnp.float32),
            pltpu.VMEM((TM, cols), jnp.float32),
            pltpu.VMEM((TM, cols), jnp.float32),
        ],
        compiler_params=pltpu.CompilerParams(
            dimension_semantics=("arbitrary",)),
    )(co33, padded_flat, padded3, s_full, dinv, bias, w2b)


def kernel(x, edge_index, w1, b1, w2, b2):
    n, f_in = x.shape
    hid = w1.shape[1]
    f_out = w2.shape[1]

    n_pad = _round_up(n, TM)
    f_in_pad = _round_up(f_in, LANE)
    hid_pad = _round_up(hid, LANE)
    f_out_pad = _round_up(f_out, LANE)
    nblk = n_pad // TM

    src = edge_index[0].astype(jnp.int32)
    dst = edge_index[1].astype(jnp.int32)
    ne = src.shape[0]
    er = _round_up(ne, LANE)
    eb = er // LANE
    p_max = _round_up(ne, TM) + nblk * TM
    nc_max = p_max // TM

    key = jnp.pad(dst // TM, (0, er - ne), constant_values=-1)
    pos3, co2 = _pos(key.reshape(eb, 1, LANE), nblk)
    co = co2.reshape(-1)

    pos = pos3.reshape(-1)
    valid = jnp.arange(er, dtype=jnp.int32) < ne
    pos = jnp.where(valid, pos, p_max)

    dstl = dst % TM
    packed = jnp.pad(src, (0, er - ne)) | (jnp.pad(dstl, (0, er - ne))
                                           << SRC_BITS)
    padded = jnp.full((p_max,), SENT, jnp.int32).at[pos].set(packed)
    padded3 = padded.reshape(nc_max, 1, TM)

    x_p = _pad2(x, n_pad, f_in_pad)
    w1b = _pad2(w1, f_in_pad, hid_pad).astype(jnp.bfloat16)
    w2b = _pad2(w2, hid_pad, f_out_pad).astype(jnp.bfloat16)
    b1_p = _pad2(b1.reshape(1, -1), 1, hid_pad)
    b2_p = _pad2(b2.reshape(1, -1), 1, f_out_pad)

    dinv = _deg(co, padded3, n_pad, n)
    s1 = _proj(x_p, w1b, dinv)
    m2 = _agg(co, padded, padded3, s1, dinv, b1_p, w2b, last=False,
              out_cols=f_out_pad)
    out_p = _agg(co, padded, padded3, m2, dinv, b2_p, w2b, last=True,
                 out_cols=f_out_pad)

    return out_p[:n, :f_out]


# speedup vs baseline: 1.8382x; 1.3078x over previous
import functools

import jax
import jax.numpy as jnp
from jax.experimental import pallas as pl
from jax.experimental.pallas import tpu as pltpu


LANE = 128
TM = 256
SRC_BITS = 13
SENT = 1 << 22


def _round_up(x, m):
    return (x + m - 1) // m * m


def _pad2(a, rows, cols):
    pr, pc = rows - a.shape[0], cols - a.shape[1]
    if pr == 0 and pc == 0:
        return a
    return jnp.pad(a, ((0, pr), (0, pc)))


def _deg_kernel(co_ref, pad_ref, o_ref, *, n):
    i = pl.program_id(0)
    o_ref[...] = jnp.zeros_like(o_ref)
    c0 = co_ref[i]
    c1 = co_ref[i + 1]

    def chunk(c, _):
        dstl = pad_ref[pl.ds(c, 1), 0, :] >> SRC_BITS
        iot = jax.lax.broadcasted_iota(jnp.int32, (TM, TM), 0)
        dt = jnp.where(iot == dstl, 1.0, 0.0)
        o_ref[...] += jnp.sum(dt, axis=1, keepdims=True)
        return 0

    jax.lax.fori_loop(c0, c1, chunk, 0)

    row = i * TM + jax.lax.broadcasted_iota(jnp.int32, (TM, 1), 0)
    deg = o_ref[...] + 1.0
    o_ref[...] = jnp.where(row < n, 1.0 / jnp.sqrt(deg), 0.0)


def _proj_kernel(x_ref, w_ref, d_ref, o_ref):
    xb = x_ref[...].astype(jnp.bfloat16)
    acc = jnp.dot(xb, w_ref[...], preferred_element_type=jnp.float32)
    o_ref[...] = acc * d_ref[...]


def _agg_kernel(co_ref, src_sm_ref, pad_vm_ref, s_ref, d_ref, b_ref, w2_ref,
                o_ref, acc_ref, g_ref, g2_ref, g3_ref, g4_ref, *, last):
    i = pl.program_id(0)
    acc_ref[...] = jnp.zeros_like(acc_ref)
    c0 = co_ref[i]
    c1 = co_ref[i + 1]

    iot = jax.lax.broadcasted_iota(jnp.int32, (TM, TM), 0)

    def onehot(c):
        dstl = pad_vm_ref[pl.ds(c, 1), 0, :] >> SRC_BITS
        return jnp.where(iot == dstl, 1.0, 0.0).astype(jnp.float32)

    grefs = (g_ref, g2_ref, g3_ref, g4_ref)

    def quad(j, _):
        cbs = [(c0 + 4 * j + q) * TM for q in range(4)]
        for e in range(TM):
            for q in range(4):
                s0 = src_sm_ref[cbs[q] + e]
                grefs[q][pl.ds(e, 1), :] = s_ref[pl.ds(s0, 1), :]
        for q in range(4):
            acc_ref[...] += jnp.dot(onehot(c0 + 4 * j + q), grefs[q][...],
                                    preferred_element_type=jnp.float32)
        return 0

    jax.lax.fori_loop(0, (c1 - c0) // 4, quad, 0)

    def tail(c, _):
        for e in range(TM):
            srcv = src_sm_ref[c * TM + e]
            g_ref[pl.ds(e, 1), :] = s_ref[pl.ds(srcv, 1), :]
        acc_ref[...] += jnp.dot(onehot(c), g_ref[...],
                                preferred_element_type=jnp.float32)
        return 0

    jax.lax.fori_loop(c0 + ((c1 - c0) // 4) * 4, c1, tail, 0)

    acc = acc_ref[...] + s_ref[pl.ds(i * TM, TM), :]
    if last:
        o_ref[...] = acc * d_ref[...] + b_ref[...]
    else:
        h = jnp.maximum(acc * d_ref[...] + b_ref[...], 0.0)
        m2 = jnp.dot(h.astype(jnp.bfloat16), w2_ref[...],
                     preferred_element_type=jnp.float32)
        o_ref[...] = m2 * d_ref[...]


def _deg(co33, padded3, n_pad, n):
    nc = padded3.shape[0]
    return pl.pallas_call(
        functools.partial(_deg_kernel, n=n),
        out_shape=jax.ShapeDtypeStruct((n_pad, 1), jnp.float32),
        grid=(n_pad // TM,),
        in_specs=[
            pl.BlockSpec(memory_space=pltpu.SMEM),
            pl.BlockSpec((nc, 1, TM), lambda i: (0, 0, 0)),
        ],
        out_specs=pl.BlockSpec((TM, 1), lambda i: (i, 0)),
        compiler_params=pltpu.CompilerParams(
            dimension_semantics=("arbitrary",)),
    )(co33, padded3)


def _proj(x_p, w1b, dinv):
    n_pad, f_in_pad = x_p.shape
    hid_pad = w1b.shape[1]
    return pl.pallas_call(
        _proj_kernel,
        out_shape=jax.ShapeDtypeStruct((n_pad, hid_pad), jnp.float32),
        grid=(n_pad // TM,),
        in_specs=[
            pl.BlockSpec((TM, f_in_pad), lambda i: (i, 0)),
            pl.BlockSpec((f_in_pad, hid_pad), lambda i: (0, 0)),
            pl.BlockSpec((TM, 1), lambda i: (i, 0)),
        ],
        out_specs=pl.BlockSpec((TM, hid_pad), lambda i: (i, 0)),
        compiler_params=pltpu.CompilerParams(
            dimension_semantics=("parallel",)),
    )(x_p, w1b, dinv)


def _agg(co33, src_flat, padded3, s_full, dinv, bias, w2b, *, last,
         out_cols):
    n_pad = s_full.shape[0]
    cols = s_full.shape[1]
    nc = padded3.shape[0]
    body = functools.partial(_agg_kernel, last=last)
    return pl.pallas_call(
        body,
        out_shape=jax.ShapeDtypeStruct((n_pad, out_cols), jnp.float32),
        grid=(n_pad // TM,),
        in_specs=[
            pl.BlockSpec(memory_space=pltpu.SMEM),
            pl.BlockSpec(memory_space=pltpu.SMEM),
            pl.BlockSpec((nc, 1, TM), lambda i: (0, 0, 0)),
            pl.BlockSpec((n_pad, cols), lambda i: (0, 0)),
            pl.BlockSpec((TM, 1), lambda i: (i, 0)),
            pl.BlockSpec((1, bias.shape[1]), lambda i: (0, 0)),
            pl.BlockSpec((w2b.shape[0], w2b.shape[1]), lambda i: (0, 0)),
        ],
        out_specs=pl.BlockSpec((TM, out_cols), lambda i: (i, 0)),
        scratch_shapes=[
            pltpu.VMEM((TM, cols), jnp.float32),
            pltpu.VMEM((TM, cols), jnp.float32),
            pltpu.VMEM((TM, cols), jnp.float32),
            pltpu.VMEM((TM, cols), jnp.float32),
            pltpu.VMEM((TM, cols), jnp.float32),
        ],
        compiler_params=pltpu.CompilerParams(
            dimension_semantics=("arbitrary",)),
    )(co33, src_flat, padded3, s_full, dinv, bias, w2b)


def kernel(x, edge_index, w1, b1, w2, b2):
    n, f_in = x.shape
    hid = w1.shape[1]
    f_out = w2.shape[1]

    n_pad = _round_up(n, TM)
    f_in_pad = _round_up(f_in, LANE)
    hid_pad = _round_up(hid, LANE)
    f_out_pad = _round_up(f_out, LANE)
    nblk = n_pad // TM

    src = edge_index[0].astype(jnp.int32)
    dst = edge_index[1].astype(jnp.int32)
    ne = src.shape[0]
    er = _round_up(ne, LANE)
    eb = er // LANE
    p_max = _round_up(ne, TM) + nblk * TM
    nc_max = p_max // TM

    key = jnp.pad(dst // TM, (0, er - ne), constant_values=-1)
    m = (key.reshape(eb, LANE)[None, :, :]
         == jnp.arange(nblk, dtype=jnp.int32)[:, None, None]
         ).astype(jnp.bfloat16)

    triu_in = jnp.triu(jnp.ones((LANE, LANE), jnp.bfloat16))
    p1 = jax.lax.dot_general(m, triu_in, (((2,), (0,)), ((), ())),
                             preferred_element_type=jnp.float32)
    bsum = p1[:, :, LANE - 1]
    tril_st = jnp.tril(jnp.ones((eb, eb), jnp.float32), k=-1)
    boff = jax.lax.dot_general(bsum, tril_st, (((1,), (1,)), ((), ())),
                               preferred_element_type=jnp.float32)

    sizes = bsum.sum(axis=1).astype(jnp.int32)
    nch = jnp.maximum((sizes + TM - 1) // TM, 1)
    co = jnp.concatenate([jnp.zeros(1, jnp.int32),
                          jnp.cumsum(nch, dtype=jnp.int32)])
    poff = co[:-1] * TM

    mf = m.astype(jnp.float32)
    base = boff + poff.astype(jnp.float32)[:, None]
    pos1 = ((p1 + base[:, :, None]) * mf).sum(axis=0)
    pos = pos1.reshape(-1).astype(jnp.int32) - 1
    valid = jnp.arange(er, dtype=jnp.int32) < ne
    pos = jnp.where(valid, pos, p_max)

    dstl = dst % TM
    packed = jnp.pad(src, (0, er - ne)) | (jnp.pad(dstl, (0, er - ne))
                                           << SRC_BITS)
    padded = jnp.full((p_max,), SENT, jnp.int32).at[pos].set(packed)
    padded3 = padded.reshape(nc_max, 1, TM)
    src_flat = padded & ((1 << SRC_BITS) - 1)

    x_p = _pad2(x, n_pad, f_in_pad)
    w1b = _pad2(w1, f_in_pad, hid_pad).astype(jnp.bfloat16)
    w2b = _pad2(w2, hid_pad, f_out_pad).astype(jnp.bfloat16)
    b1_p = _pad2(b1.reshape(1, -1), 1, hid_pad)
    b2_p = _pad2(b2.reshape(1, -1), 1, f_out_pad)

    dinv = _deg(co, padded3, n_pad, n)
    s1 = _proj(x_p, w1b, dinv)
    m2 = _agg(co, src_flat, padded3, s1, dinv, b1_p, w2b, last=False,
              out_cols=f_out_pad)
    out_p = _agg(co, src_flat, padded3, m2, dinv, b2_p, w2b, last=True,
                 out_cols=f_out_pad)

    return out_p[:n, :f_out]


# speedup vs baseline: 1.8666x; 1.0155x over previous
import functools

import jax
import jax.numpy as jnp
from jax.experimental import pallas as pl
from jax.experimental.pallas import tpu as pltpu


LANE = 128
TM = 256
SRC_BITS = 13
SENT = 1 << 22


def _round_up(x, m):
    return (x + m - 1) // m * m


def _pad2(a, rows, cols):
    pr, pc = rows - a.shape[0], cols - a.shape[1]
    if pr == 0 and pc == 0:
        return a
    return jnp.pad(a, ((0, pr), (0, pc)))


def _deg_kernel(co_ref, pad_ref, o_ref, *, n):
    i = pl.program_id(0)
    o_ref[...] = jnp.zeros_like(o_ref)
    c0 = co_ref[i]
    c1 = co_ref[i + 1]

    def chunk(c, _):
        dstl = pad_ref[pl.ds(c, 1), 0, :] >> SRC_BITS
        iot = jax.lax.broadcasted_iota(jnp.int32, (TM, TM), 0)
        dt = jnp.where(iot == dstl, 1.0, 0.0)
        o_ref[...] += jnp.sum(dt, axis=1, keepdims=True)
        return 0

    jax.lax.fori_loop(c0, c1, chunk, 0)

    row = i * TM + jax.lax.broadcasted_iota(jnp.int32, (TM, 1), 0)
    deg = o_ref[...] + 1.0
    o_ref[...] = jnp.where(row < n, 1.0 / jnp.sqrt(deg), 0.0)


def _proj_kernel(x_ref, w_ref, d_ref, o_ref):
    xb = x_ref[...].astype(jnp.bfloat16)
    acc = jnp.dot(xb, w_ref[...], preferred_element_type=jnp.float32)
    o_ref[...] = acc * d_ref[...]


def _agg_kernel(co_ref, src_sm_ref, pad_vm_ref, s_ref, d_ref, b_ref, w2_ref,
                o_ref, acc_ref, g_ref, g2_ref, g3_ref, g4_ref, g5_ref,
                g6_ref, g7_ref, g8_ref, *, last):
    i = pl.program_id(0)
    acc_ref[...] = jnp.zeros_like(acc_ref)
    c0 = co_ref[i]
    c1 = co_ref[i + 1]

    iot = jax.lax.broadcasted_iota(jnp.int32, (TM, TM), 0)

    def onehot(c):
        dstl = pad_vm_ref[pl.ds(c, 1), 0, :] >> SRC_BITS
        return jnp.where(iot == dstl, 1.0, 0.0).astype(jnp.float32)

    grefs = (g_ref, g2_ref, g3_ref, g4_ref, g5_ref, g6_ref, g7_ref, g8_ref)
    NW = 8

    def group(j, _):
        cbs = [(c0 + NW * j + q) * TM for q in range(NW)]
        for e in range(TM):
            for q in range(NW):
                s0 = src_sm_ref[cbs[q] + e]
                grefs[q][pl.ds(e, 1), :] = s_ref[pl.ds(s0, 1), :]
        for q in range(NW):
            acc_ref[...] += jnp.dot(onehot(c0 + NW * j + q), grefs[q][...],
                                    preferred_element_type=jnp.float32)
        return 0

    jax.lax.fori_loop(0, (c1 - c0) // NW, group, 0)

    def tail(c, _):
        for e in range(TM):
            srcv = src_sm_ref[c * TM + e]
            g_ref[pl.ds(e, 1), :] = s_ref[pl.ds(srcv, 1), :]
        acc_ref[...] += jnp.dot(onehot(c), g_ref[...],
                                preferred_element_type=jnp.float32)
        return 0

    jax.lax.fori_loop(c0 + ((c1 - c0) // NW) * NW, c1, tail, 0)

    acc = acc_ref[...] + s_ref[pl.ds(i * TM, TM), :]
    if last:
        o_ref[...] = acc * d_ref[...] + b_ref[...]
    else:
        h = jnp.maximum(acc * d_ref[...] + b_ref[...], 0.0)
        m2 = jnp.dot(h.astype(jnp.bfloat16), w2_ref[...],
                     preferred_element_type=jnp.float32)
        o_ref[...] = m2 * d_ref[...]


def _deg(co33, padded3, n_pad, n):
    nc = padded3.shape[0]
    return pl.pallas_call(
        functools.partial(_deg_kernel, n=n),
        out_shape=jax.ShapeDtypeStruct((n_pad, 1), jnp.float32),
        grid=(n_pad // TM,),
        in_specs=[
            pl.BlockSpec(memory_space=pltpu.SMEM),
            pl.BlockSpec((nc, 1, TM), lambda i: (0, 0, 0)),
        ],
        out_specs=pl.BlockSpec((TM, 1), lambda i: (i, 0)),
        compiler_params=pltpu.CompilerParams(
            dimension_semantics=("arbitrary",)),
    )(co33, padded3)


def _proj(x_p, w1b, dinv):
    n_pad, f_in_pad = x_p.shape
    hid_pad = w1b.shape[1]
    return pl.pallas_call(
        _proj_kernel,
        out_shape=jax.ShapeDtypeStruct((n_pad, hid_pad), jnp.float32),
        grid=(n_pad // TM,),
        in_specs=[
            pl.BlockSpec((TM, f_in_pad), lambda i: (i, 0)),
            pl.BlockSpec((f_in_pad, hid_pad), lambda i: (0, 0)),
            pl.BlockSpec((TM, 1), lambda i: (i, 0)),
        ],
        out_specs=pl.BlockSpec((TM, hid_pad), lambda i: (i, 0)),
        compiler_params=pltpu.CompilerParams(
            dimension_semantics=("parallel",)),
    )(x_p, w1b, dinv)


def _agg(co33, src_flat, padded3, s_full, dinv, bias, w2b, *, last,
         out_cols):
    n_pad = s_full.shape[0]
    cols = s_full.shape[1]
    nc = padded3.shape[0]
    body = functools.partial(_agg_kernel, last=last)
    return pl.pallas_call(
        body,
        out_shape=jax.ShapeDtypeStruct((n_pad, out_cols), jnp.float32),
        grid=(n_pad // TM,),
        in_specs=[
            pl.BlockSpec(memory_space=pltpu.SMEM),
            pl.BlockSpec(memory_space=pltpu.SMEM),
            pl.BlockSpec((nc, 1, TM), lambda i: (0, 0, 0)),
            pl.BlockSpec((n_pad, cols), lambda i: (0, 0)),
            pl.BlockSpec((TM, 1), lambda i: (i, 0)),
            pl.BlockSpec((1, bias.shape[1]), lambda i: (0, 0)),
            pl.BlockSpec((w2b.shape[0], w2b.shape[1]), lambda i: (0, 0)),
        ],
        out_specs=pl.BlockSpec((TM, out_cols), lambda i: (i, 0)),
        scratch_shapes=[
            pltpu.VMEM((TM, cols), jnp.float32),
            pltpu.VMEM((TM, cols), jnp.float32),
            pltpu.VMEM((TM, cols), jnp.float32),
            pltpu.VMEM((TM, cols), jnp.float32),
            pltpu.VMEM((TM, cols), jnp.float32),
            pltpu.VMEM((TM, cols), jnp.float32),
            pltpu.VMEM((TM, cols), jnp.float32),
            pltpu.VMEM((TM, cols), jnp.float32),
            pltpu.VMEM((TM, cols), jnp.float32),
        ],
        compiler_params=pltpu.CompilerParams(
            dimension_semantics=("arbitrary",)),
    )(co33, src_flat, padded3, s_full, dinv, bias, w2b)


def kernel(x, edge_index, w1, b1, w2, b2):
    n, f_in = x.shape
    hid = w1.shape[1]
    f_out = w2.shape[1]

    n_pad = _round_up(n, TM)
    f_in_pad = _round_up(f_in, LANE)
    hid_pad = _round_up(hid, LANE)
    f_out_pad = _round_up(f_out, LANE)
    nblk = n_pad // TM

    src = edge_index[0].astype(jnp.int32)
    dst = edge_index[1].astype(jnp.int32)
    ne = src.shape[0]
    er = _round_up(ne, LANE)
    eb = er // LANE
    p_max = _round_up(ne, TM) + nblk * TM
    nc_max = p_max // TM

    key = jnp.pad(dst // TM, (0, er - ne), constant_values=-1)
    m = (key.reshape(eb, LANE)[None, :, :]
         == jnp.arange(nblk, dtype=jnp.int32)[:, None, None]
         ).astype(jnp.bfloat16)

    triu_in = jnp.triu(jnp.ones((LANE, LANE), jnp.bfloat16))
    p1 = jax.lax.dot_general(m, triu_in, (((2,), (0,)), ((), ())),
                             preferred_element_type=jnp.float32)
    bsum = p1[:, :, LANE - 1]
    tril_st = jnp.tril(jnp.ones((eb, eb), jnp.float32), k=-1)
    boff = jax.lax.dot_general(bsum, tril_st, (((1,), (1,)), ((), ())),
                               preferred_element_type=jnp.float32)

    sizes = bsum.sum(axis=1).astype(jnp.int32)
    nch = jnp.maximum((sizes + TM - 1) // TM, 1)
    co = jnp.concatenate([jnp.zeros(1, jnp.int32),
                          jnp.cumsum(nch, dtype=jnp.int32)])
    poff = co[:-1] * TM

    mf = m.astype(jnp.float32)
    base = boff + poff.astype(jnp.float32)[:, None]
    pos1 = ((p1 + base[:, :, None]) * mf).sum(axis=0)
    pos = pos1.reshape(-1).astype(jnp.int32) - 1
    valid = jnp.arange(er, dtype=jnp.int32) < ne
    pos = jnp.where(valid, pos, p_max)

    dstl = dst % TM
    packed = jnp.pad(src, (0, er - ne)) | (jnp.pad(dstl, (0, er - ne))
                                           << SRC_BITS)
    padded = jnp.full((p_max,), SENT, jnp.int32).at[pos].set(packed)
    padded3 = padded.reshape(nc_max, 1, TM)
    src_flat = padded & ((1 << SRC_BITS) - 1)

    x_p = _pad2(x, n_pad, f_in_pad)
    w1b = _pad2(w1, f_in_pad, hid_pad).astype(jnp.bfloat16)
    w2b = _pad2(w2, hid_pad, f_out_pad).astype(jnp.bfloat16)
    b1_p = _pad2(b1.reshape(1, -1), 1, hid_pad)
    b2_p = _pad2(b2.reshape(1, -1), 1, f_out_pad)

    dinv = _deg(co, padded3, n_pad, n)
    s1 = _proj(x_p, w1b, dinv)
    m2 = _agg(co, src_flat, padded3, s1, dinv, b1_p, w2b, last=False,
              out_cols=f_out_pad)
    out_p = _agg(co, src_flat, padded3, m2, dinv, b2_p, w2b, last=True,
                 out_cols=f_out_pad)

    return out_p[:n, :f_out]
